# baseline (speedup 1.0000x reference)
import jax
import jax.numpy as jnp
from jax import lax
from jax.experimental import pallas as pl
from jax.experimental.pallas import tpu as pltpu
from jax.experimental.pallas import tpu_sc as plsc

B = 1024
L = 200
N = B * L
WORD_DIM = 128
SMALL_DIM = 16
OUT_DIM = WORD_DIM + 3 * SMALL_DIM

NC = 2
NS = 16
NW = NC * NS
N_PER_W = N // NW
CHUNK = 128
M = N_PER_W // CHUNK
K = 4
D = 2


def _sc_body(idx_hbm,
             word_tbl, tag_tbl, pos1_tbl, pos2_tbl,
             out_hbm,
             idxb, wb, tb, p1b, p2b, gsems, wsems, isems):
  wid = lax.axis_index("s") * NC + lax.axis_index("c")
  wbase = wid * N_PER_W
  cbase = wid * M

  def fire_idx(t, b):
    pltpu.async_copy(idx_hbm.at[pl.ds((cbase + t) * 4 * CHUNK, 4 * CHUNK)], idxb[b], isems[b])

  def fire_gathers(t, b):
    del t
    pltpu.make_async_copy(idx_hbm.at[pl.ds(cbase * 4 * CHUNK, 4 * CHUNK)], idxb[b], isems[b]).wait()
    pltpu.async_copy(word_tbl.at[idxb[b].at[pl.ds(0, CHUNK)]], wb[b], gsems[b])
    pltpu.async_copy(tag_tbl.at[idxb[b].at[pl.ds(CHUNK, CHUNK)]], tb[b], gsems[b])
    pltpu.async_copy(pos1_tbl.at[idxb[b].at[pl.ds(2 * CHUNK, CHUNK)]], p1b[b], gsems[b])
    pltpu.async_copy(pos2_tbl.at[idxb[b].at[pl.ds(3 * CHUNK, CHUNK)]], p2b[b], gsems[b])

  def wait_gathers(b):
    pltpu.make_async_copy(word_tbl.at[idxb[b].at[pl.ds(0, CHUNK)]], wb[b], gsems[b]).wait()
    pltpu.make_async_copy(tag_tbl.at[idxb[b].at[pl.ds(CHUNK, CHUNK)]], tb[b], gsems[b]).wait()
    pltpu.make_async_copy(pos1_tbl.at[idxb[b].at[pl.ds(2 * CHUNK, CHUNK)]], p1b[b], gsems[b]).wait()
    pltpu.make_async_copy(pos2_tbl.at[idxb[b].at[pl.ds(3 * CHUNK, CHUNK)]], p2b[b], gsems[b]).wait()

  def fire_writes(t, b):
    base = wbase + t * CHUNK
    pltpu.async_copy(wb[b], out_hbm.at[pl.ds(base, CHUNK), pl.ds(0, WORD_DIM)], wsems[b])
    pltpu.async_copy(tb[b], out_hbm.at[pl.ds(base, CHUNK), pl.ds(128, SMALL_DIM)], wsems[b])
    pltpu.async_copy(p1b[b], out_hbm.at[pl.ds(base, CHUNK), pl.ds(144, SMALL_DIM)], wsems[b])
    pltpu.async_copy(p2b[b], out_hbm.at[pl.ds(base, CHUNK), pl.ds(160, SMALL_DIM)], wsems[b])

  def wait_writes(b):
    pltpu.make_async_copy(wb[b], out_hbm.at[pl.ds(wbase, CHUNK), pl.ds(0, WORD_DIM)], wsems[b]).wait()
    pltpu.make_async_copy(tb[b], out_hbm.at[pl.ds(wbase, CHUNK), pl.ds(128, SMALL_DIM)], wsems[b]).wait()
    pltpu.make_async_copy(p1b[b], out_hbm.at[pl.ds(wbase, CHUNK), pl.ds(144, SMALL_DIM)], wsems[b]).wait()
    pltpu.make_async_copy(p2b[b], out_hbm.at[pl.ds(wbase, CHUNK), pl.ds(160, SMALL_DIM)], wsems[b]).wait()


  for b in range(K):
    fire_idx(b, b)

  for t in range(K):
    if t - D >= 0:
      bw = (t - D) % K
      wait_gathers(bw)
      if t - D + K < M:
        fire_idx(t - D + K, bw)
      fire_writes(t - D, bw)
    fire_gathers(t, t % K)

  n_groups = (M - K) // K

  def group(g, _):
    for u in range(K):
      t = K + g * K + u
      bw = (K + u - D) % K
      wait_gathers(bw)
      fire_idx(t - D + K, bw)
      fire_writes(t - D, bw)
      br = u % K
      wait_writes(br)
      fire_gathers(t, br)
    return ()

  lax.fori_loop(0, n_groups, group, ())

  for t in range(K + n_groups * K, M):
    bw = (t - D) % K
    wait_gathers(bw)
    if t - D + K < M:
      fire_idx(t - D + K, bw)
    fire_writes(t - D, bw)
    br = t % K
    wait_writes(br)
    fire_gathers(t, br)
  for t in range(M, M + D):
    bw = (t - D) % K
    wait_gathers(bw)
    fire_writes(t - D, bw)
  for i in range(K):
    wait_writes((M - 1 - i) % K)


def _sc_kernel_fn():
  mesh = plsc.VectorSubcoreMesh(core_axis_name="c", subcore_axis_name="s")

  def body(idx_hbm, wt, tt, p1t, p2t, out_hbm, *scratch):
    idxb = scratch[0:K]
    wb = scratch[K:2 * K]
    tb = scratch[2 * K:3 * K]
    p1b = scratch[3 * K:4 * K]
    p2b = scratch[4 * K:5 * K]
    gsems = scratch[5 * K:6 * K]
    wsems = scratch[6 * K:7 * K]
    isems = scratch[7 * K:8 * K]
    _sc_body(idx_hbm, wt, tt, p1t, p2t, out_hbm,
             idxb, wb, tb, p1b, p2b, gsems, wsems, isems)

  scratch_types = (
      [pltpu.VMEM((4 * CHUNK,), jnp.int32) for _ in range(K)]
      + [pltpu.VMEM((CHUNK, WORD_DIM), jnp.float32) for _ in range(K)]
      + [pltpu.VMEM((CHUNK, SMALL_DIM), jnp.float32) for _ in range(K)]
      + [pltpu.VMEM((CHUNK, SMALL_DIM), jnp.float32) for _ in range(K)]
      + [pltpu.VMEM((CHUNK, SMALL_DIM), jnp.float32) for _ in range(K)]
      + [pltpu.SemaphoreType.DMA for _ in range(K)]
      + [pltpu.SemaphoreType.DMA for _ in range(K)]
      + [pltpu.SemaphoreType.DMA for _ in range(K)]
  )
  return pl.kernel(
      body,
      out_type=jax.ShapeDtypeStruct((N, OUT_DIM), jnp.float32),
      mesh=mesh,
      scratch_types=scratch_types,
      compiler_params=pltpu.CompilerParams(use_tc_tiling_on_sc=False),
  )


@jax.jit
def _embed(word, tag, pos1, pos2, word_tbl, tag_tbl, pos1_tbl, pos2_tbl):
  idx = jnp.stack([
      word.reshape(N // CHUNK, CHUNK).astype(jnp.int32),
      tag.reshape(N // CHUNK, CHUNK).astype(jnp.int32),
      pos1.reshape(N // CHUNK, CHUNK).astype(jnp.int32),
      pos2.reshape(N // CHUNK, CHUNK).astype(jnp.int32),
  ], axis=1).reshape(4 * N)
  out = _sc_kernel_fn()(idx, word_tbl, tag_tbl, pos1_tbl, pos2_tbl)
  return out.reshape(B, L, OUT_DIM)


def kernel(word, tag, pos1, pos2, word_table, tag_table, pos1_table, pos2_table):
  return _embed(word, tag, pos1, pos2,
                word_table, tag_table, pos1_table, pos2_table)


# speedup vs baseline: 1.6107x; 1.6107x over previous
import jax
import jax.numpy as jnp
from jax import lax
from jax.experimental import pallas as pl
from jax.experimental.pallas import tpu as pltpu
from jax.experimental.pallas import tpu_sc as plsc

B = 1024
L = 200
N = B * L
WORD_DIM = 128
SMALL_DIM = 16
SMALL_ROWS = 30 + 512 + 512
OUT_DIM = WORD_DIM + 3 * SMALL_DIM

NC = 2
NS = 16
NW = NC * NS
N_PER_W = N // NW
CHUNK = 128
M = N_PER_W // CHUNK
KB = 3


def _sc_body(idx_hbm, word_tbl, small_tbl_hbm, out_hbm,
             idxb, wb, sbuf, stbl, gsems, isems, tsem):
  wid = lax.axis_index("s") * NC + lax.axis_index("c")
  wbase = wid * N_PER_W
  cbase = wid * M
  lanes = lax.iota(jnp.int32, 16)

  pltpu.async_copy(small_tbl_hbm, stbl, tsem).wait()

  def fire_stage(i, b):
    pltpu.async_copy(idx_hbm.at[pl.ds((cbase + i) * 4 * CHUNK, 4 * CHUNK)], idxb[b], isems[b])

  def wait_stage(b):
    pltpu.make_async_copy(idx_hbm.at[pl.ds(cbase * 4 * CHUNK, 4 * CHUNK)], idxb[b], isems[b]).wait()

  def fire_word(b):
    pltpu.async_copy(word_tbl.at[idxb[b].at[pl.ds(0, CHUNK)]], wb[b], gsems[b])

  def wait_word(b):
    pltpu.make_async_copy(word_tbl.at[idxb[b].at[pl.ds(0, CHUNK)]], wb[b], gsems[b]).wait()

  def compute_smalls(b):
    for t in range(1, 4):
      col = (t - 1) * SMALL_DIM

      def grp(g, _):
        v = idxb[b][pl.ds(t * CHUNK + g * 16, 16)]
        for jj in range(16):
          row = plsc.load_gather(stbl, [v[jj] + lanes])
          sbuf[g * 16 + jj, col:col + SMALL_DIM] = row
        return ()

      lax.fori_loop(0, CHUNK // 16, grp, ())

  def write_out(i, b):
    base = wbase + i * CHUNK
    pltpu.sync_copy(wb[b], out_hbm.at[pl.ds(base, CHUNK), pl.ds(0, WORD_DIM)])
    pltpu.sync_copy(sbuf, out_hbm.at[pl.ds(base, CHUNK), pl.ds(WORD_DIM, 3 * SMALL_DIM)])

  def step(i, b, fire_g, fire_i):
    if fire_g:
      wait_stage((b + 1) % KB)
      fire_word((b + 1) % KB)
    if fire_i:
      fire_stage(i + 2, (b + 2) % KB)
    compute_smalls(b)
    wait_word(b)
    write_out(i, b)

  fire_stage(0, 0)
  wait_stage(0)
  fire_word(0)
  fire_stage(1, 1)

  n_groups = (M - 2) // KB

  def group(g, _):
    for u in range(KB):
      step(g * KB + u, u, True, True)
    return ()

  lax.fori_loop(0, n_groups, group, ())

  for i in range(n_groups * KB, M):
    step(i, i % KB, i + 1 < M, i + 2 < M)


def _sc_kernel_fn():
  mesh = plsc.VectorSubcoreMesh(core_axis_name="c", subcore_axis_name="s")

  def body(idx_hbm, wt, st, out_hbm, *scratch):
    idxb = scratch[0:KB]
    wb = scratch[KB:2 * KB]
    sbuf, stbl = scratch[2 * KB], scratch[2 * KB + 1]
    gsems = scratch[2 * KB + 2:3 * KB + 2]
    isems = scratch[3 * KB + 2:4 * KB + 2]
    tsem = scratch[4 * KB + 2]
    _sc_body(idx_hbm, wt, st, out_hbm, idxb, wb, sbuf, stbl, gsems, isems, tsem)

  scratch_types = (
      [pltpu.VMEM((4 * CHUNK,), jnp.int32) for _ in range(KB)]
      + [pltpu.VMEM((CHUNK, WORD_DIM), jnp.float32) for _ in range(KB)]
      + [pltpu.VMEM((CHUNK, 3 * SMALL_DIM), jnp.float32),
         pltpu.VMEM((SMALL_ROWS * SMALL_DIM,), jnp.float32)]
      + [pltpu.SemaphoreType.DMA for _ in range(KB)]
      + [pltpu.SemaphoreType.DMA for _ in range(KB)]
      + [pltpu.SemaphoreType.DMA]
  )
  return pl.kernel(
      body,
      out_type=jax.ShapeDtypeStruct((N, OUT_DIM), jnp.float32),
      mesh=mesh,
      scratch_types=scratch_types,
      compiler_params=pltpu.CompilerParams(
          use_tc_tiling_on_sc=False, needs_layout_passes=False),
  )


@jax.jit
def _embed(word, tag, pos1, pos2, word_tbl, tag_tbl, pos1_tbl, pos2_tbl):
  nb = N // CHUNK
  idx = jnp.stack([
      word.reshape(nb, CHUNK).astype(jnp.int32),
      tag.reshape(nb, CHUNK).astype(jnp.int32) * SMALL_DIM,
      (pos1.reshape(nb, CHUNK).astype(jnp.int32) + 30) * SMALL_DIM,
      (pos2.reshape(nb, CHUNK).astype(jnp.int32) + 542) * SMALL_DIM,
  ], axis=1).reshape(4 * N)
  small_tbl = jnp.concatenate(
      [tag_tbl, pos1_tbl, pos2_tbl], axis=0).reshape(SMALL_ROWS * SMALL_DIM)
  out = _sc_kernel_fn()(idx, word_tbl, small_tbl)
  return out.reshape(B, L, OUT_DIM)


def kernel(word, tag, pos1, pos2, word_table, tag_table, pos1_table, pos2_table):
  return _embed(word, tag, pos1, pos2,
                word_table, tag_table, pos1_table, pos2_table)


# speedup vs baseline: 1.6330x; 1.0138x over previous
import jax
import jax.numpy as jnp
from jax import lax
from jax.experimental import pallas as pl
from jax.experimental.pallas import tpu as pltpu
from jax.experimental.pallas import tpu_sc as plsc

B = 1024
L = 200
N = B * L
WORD_DIM = 128
SMALL_DIM = 16
SMALL_ROWS = 30 + 512 + 512
OUT_DIM = WORD_DIM + 3 * SMALL_DIM

NC = 2
NS = 16
NW = NC * NS
N_PER_W = N // NW
CHUNK = 128
M = N_PER_W // CHUNK
KB = 3


def _sc_body(idx_hbm, word_tbl, small_tbl_hbm, word_out, small_out,
             idxb, wb, sbuf, stbl, gsems, isems, tsem):
  wid = lax.axis_index("s") * NC + lax.axis_index("c")
  wbase = wid * N_PER_W
  cbase = wid * M
  lanes = lax.iota(jnp.int32, 16)

  pltpu.async_copy(small_tbl_hbm, stbl, tsem).wait()

  def fire_stage(i, b):
    pltpu.async_copy(idx_hbm.at[pl.ds((cbase + i) * 4 * CHUNK, 4 * CHUNK)], idxb[b], isems[b])

  def wait_stage(b):
    pltpu.make_async_copy(idx_hbm.at[pl.ds(cbase * 4 * CHUNK, 4 * CHUNK)], idxb[b], isems[b]).wait()

  def fire_word(b):
    pltpu.async_copy(word_tbl.at[idxb[b].at[pl.ds(0, CHUNK)]], wb[b], gsems[b])

  def wait_word(b):
    pltpu.make_async_copy(word_tbl.at[idxb[b].at[pl.ds(0, CHUNK)]], wb[b], gsems[b]).wait()

  def compute_smalls(b):
    for t in range(1, 4):
      col = (t - 1) * SMALL_DIM

      def grp(g, _):
        v = idxb[b][pl.ds(t * CHUNK + g * 16, 16)]
        for jj in range(16):
          row = plsc.load_gather(stbl, [v[jj] + lanes])
          sbuf[g * 16 + jj, col:col + SMALL_DIM] = row
        return ()

      lax.fori_loop(0, CHUNK // 16, grp, ())

  def write_out(i, b):
    base = wbase + i * CHUNK
    pltpu.sync_copy(wb[b], word_out.at[pl.ds(base, CHUNK), :])
    pltpu.sync_copy(sbuf, small_out.at[pl.ds(base, CHUNK), :])

  def step(i, b, fire_g, fire_i):
    if fire_g:
      wait_stage((b + 1) % KB)
      fire_word((b + 1) % KB)
    if fire_i:
      fire_stage(i + 2, (b + 2) % KB)
    compute_smalls(b)
    wait_word(b)
    write_out(i, b)

  fire_stage(0, 0)
  wait_stage(0)
  fire_word(0)
  fire_stage(1, 1)

  n_groups = (M - 2) // KB

  def group(g, _):
    for u in range(KB):
      step(g * KB + u, u, True, True)
    return ()

  lax.fori_loop(0, n_groups, group, ())

  for i in range(n_groups * KB, M):
    step(i, i % KB, i + 1 < M, i + 2 < M)


def _sc_kernel_fn():
  mesh = plsc.VectorSubcoreMesh(core_axis_name="c", subcore_axis_name="s")

  def body(idx_hbm, wt, st, word_out, small_out, *scratch):
    idxb = scratch[0:KB]
    wb = scratch[KB:2 * KB]
    sbuf, stbl = scratch[2 * KB], scratch[2 * KB + 1]
    gsems = scratch[2 * KB + 2:3 * KB + 2]
    isems = scratch[3 * KB + 2:4 * KB + 2]
    tsem = scratch[4 * KB + 2]
    _sc_body(idx_hbm, wt, st, word_out, small_out,
             idxb, wb, sbuf, stbl, gsems, isems, tsem)

  scratch_types = (
      [pltpu.VMEM((4 * CHUNK,), jnp.int32) for _ in range(KB)]
      + [pltpu.VMEM((CHUNK, WORD_DIM), jnp.float32) for _ in range(KB)]
      + [pltpu.VMEM((CHUNK, WORD_DIM), jnp.float32),
         pltpu.VMEM((SMALL_ROWS * SMALL_DIM,), jnp.float32)]
      + [pltpu.SemaphoreType.DMA for _ in range(KB)]
      + [pltpu.SemaphoreType.DMA for _ in range(KB)]
      + [pltpu.SemaphoreType.DMA]
  )
  return pl.kernel(
      body,
      out_type=(jax.ShapeDtypeStruct((N, WORD_DIM), jnp.float32),
                jax.ShapeDtypeStruct((N, WORD_DIM), jnp.float32)),
      mesh=mesh,
      scratch_types=scratch_types,
      compiler_params=pltpu.CompilerParams(
          use_tc_tiling_on_sc=False, needs_layout_passes=False),
  )


@jax.jit
def _embed(word, tag, pos1, pos2, word_tbl, tag_tbl, pos1_tbl, pos2_tbl):
  nb = N // CHUNK
  idx = jnp.stack([
      word.reshape(nb, CHUNK).astype(jnp.int32),
      tag.reshape(nb, CHUNK).astype(jnp.int32) * SMALL_DIM,
      (pos1.reshape(nb, CHUNK).astype(jnp.int32) + 30) * SMALL_DIM,
      (pos2.reshape(nb, CHUNK).astype(jnp.int32) + 542) * SMALL_DIM,
  ], axis=1).reshape(4 * N)
  small_tbl = jnp.concatenate(
      [tag_tbl, pos1_tbl, pos2_tbl], axis=0).reshape(SMALL_ROWS * SMALL_DIM)
  word_out, small_out = _sc_kernel_fn()(idx, word_tbl, small_tbl)
  out = _tc_assemble(word_out, small_out)
  return out.reshape(B, L, OUT_DIM)


TC_BS = 1024


def _tc_assemble_body(w_ref, s_ref, o_ref):
  o_ref[:, 0:WORD_DIM] = w_ref[...]
  o_ref[:, WORD_DIM:OUT_DIM] = s_ref[:, 0:3 * SMALL_DIM]


def _tc_assemble(word_out, small_out):
  return pl.pallas_call(
      _tc_assemble_body,
      grid=(N // TC_BS,),
      in_specs=[
          pl.BlockSpec((TC_BS, WORD_DIM), lambda j: (j, 0)),
          pl.BlockSpec((TC_BS, WORD_DIM), lambda j: (j, 0)),
      ],
      out_specs=pl.BlockSpec((TC_BS, OUT_DIM), lambda j: (j, 0)),
      out_shape=jax.ShapeDtypeStruct((N, OUT_DIM), jnp.float32),
      compiler_params=pltpu.CompilerParams(
          dimension_semantics=("arbitrary",)),
  )(word_out, small_out)


def kernel(word, tag, pos1, pos2, word_table, tag_table, pos1_table, pos2_table):
  return _embed(word, tag, pos1, pos2,
                word_table, tag_table, pos1_table, pos2_table)
